# baseline (speedup 1.0000x reference)
import functools
import math

import jax
import jax.numpy as jnp
from jax import lax
from jax.experimental import pallas as pl
from jax.experimental.pallas import tpu as pltpu
from jax.experimental.pallas import tpu_sc as plsc

TWO_PI = 2.0 * math.pi


def _make_sc_phi(batch):
    info = plsc.get_sparse_core_info()
    nc, ns, lanes = info.num_cores, info.num_subcores, info.num_lanes
    nw = nc * ns
    assert batch % (8 * nw) == 0
    b_per_w = batch // nw
    assert b_per_w % lanes == 0
    mesh = plsc.VectorSubcoreMesh(core_axis_name="c", subcore_axis_name="s")

    @functools.partial(
        pl.kernel,
        mesh=mesh,
        out_type=jax.ShapeDtypeStruct((batch,), jnp.float32),
        scratch_types=[
            pltpu.VMEM((b_per_w,), jnp.int32),
            pltpu.VMEM((b_per_w,), jnp.int32),
            pltpu.VMEM((b_per_w,), jnp.float32),
            pltpu.VMEM((b_per_w,), jnp.float32),
            pltpu.SemaphoreType.DMA,
            pltpu.SemaphoreType.DMA,
        ],
    )
    def sc_phi(x1_hbm, x2_hbm, ip_hbm, out_hbm, idx1_v, idx2_v, p1_v, p2_v,
               sem1, sem2):
        wid = lax.axis_index("s") * nc + lax.axis_index("c")
        span = pl.ds(wid * b_per_w, b_per_w)
        c1 = pltpu.async_copy(x1_hbm.at[span], idx1_v, sem1)
        c2 = pltpu.async_copy(x2_hbm.at[span], idx2_v, sem2)
        c1.wait()
        g1 = pltpu.async_copy(ip_hbm.at[idx1_v], p1_v, sem1)
        c2.wait()
        g2 = pltpu.async_copy(ip_hbm.at[idx2_v], p2_v, sem2)
        g1.wait()
        g2.wait()
        for i in range(b_per_w // lanes):
            sl = pl.ds(i * lanes, lanes)
            s = p1_v[sl] + p2_v[sl]
            p1_v[sl] = jnp.where(s >= TWO_PI, s - TWO_PI, s)
        pltpu.sync_copy(p1_v, out_hbm.at[span])

    return sc_phi


_NBUF = 4


def _tc_dense(phi, output_phases, block_rows):
    batch = phi.shape[0]
    k = output_phases.shape[0]
    nsteps = batch // block_rows
    assert nsteps % _NBUF == 0

    def body(phi_ref, op_ref, out_ref, bufs, sems):
        i = pl.program_id(0)
        n = pl.num_programs(0)
        slot = lax.rem(i, _NBUF)

        @pl.when(i >= _NBUF)
        def _():
            pltpu.make_async_copy(
                bufs.at[slot], out_ref.at[pl.ds(0, block_rows)], sems.at[slot]
            ).wait()

        d = jnp.abs(phi_ref[pl.ds(i * block_rows, block_rows), :] - op_ref[...])
        bufs[slot] = -jnp.minimum(d, TWO_PI - d)
        pltpu.make_async_copy(
            bufs.at[slot], out_ref.at[pl.ds(i * block_rows, block_rows)],
            sems.at[slot],
        ).start()

        @pl.when(i == n - 1)
        def _():
            for s in range(_NBUF):
                pltpu.make_async_copy(
                    bufs.at[s], out_ref.at[pl.ds(0, block_rows)], sems.at[s]
                ).wait()

    return pl.pallas_call(
        body,
        grid=(nsteps,),
        in_specs=[
            pl.BlockSpec((batch, 1), lambda i: (0, 0)),
            pl.BlockSpec((1, k), lambda i: (0, 0)),
        ],
        out_specs=pl.BlockSpec(memory_space=pl.ANY),
        out_shape=jax.ShapeDtypeStruct((batch, k), jnp.float32),
        scratch_shapes=[
            pltpu.VMEM((_NBUF, block_rows, k), jnp.float32),
            pltpu.SemaphoreType.DMA((_NBUF,)),
        ],
    )(phi.reshape(batch, 1), output_phases.reshape(1, k))


def kernel(x1, x2, input_phases, output_phases):
    x1 = x1.astype(jnp.int32)
    x2 = x2.astype(jnp.int32)
    input_phases = input_phases.astype(jnp.float32)
    output_phases = output_phases.astype(jnp.float32)
    batch = x1.shape[0]
    sc_phi = _make_sc_phi(batch)
    phi = sc_phi(x1, x2, input_phases)
    return _tc_dense(phi, output_phases, block_rows=2048)


# speedup vs baseline: 1.0117x; 1.0117x over previous
import functools
import math

import jax
import jax.numpy as jnp
from jax import lax
from jax.experimental import pallas as pl
from jax.experimental.pallas import tpu as pltpu
from jax.experimental.pallas import tpu_sc as plsc

TWO_PI = 2.0 * math.pi


def _make_sc_phi(batch):
    info = plsc.get_sparse_core_info()
    nc, ns, lanes = info.num_cores, info.num_subcores, info.num_lanes
    nw = nc * ns
    assert batch % (8 * nw) == 0
    b_per_w = batch // nw
    assert b_per_w % lanes == 0
    mesh = plsc.VectorSubcoreMesh(core_axis_name="c", subcore_axis_name="s")

    @functools.partial(
        pl.kernel,
        mesh=mesh,
        out_type=jax.ShapeDtypeStruct((batch,), jnp.float32),
        scratch_types=[
            pltpu.VMEM((b_per_w,), jnp.int32),
            pltpu.VMEM((b_per_w,), jnp.int32),
            pltpu.VMEM((b_per_w,), jnp.float32),
            pltpu.VMEM((b_per_w,), jnp.float32),
            pltpu.SemaphoreType.DMA,
            pltpu.SemaphoreType.DMA,
        ],
    )
    def sc_phi(x1_hbm, x2_hbm, ip_hbm, out_hbm, idx1_v, idx2_v, p1_v, p2_v,
               sem1, sem2):
        wid = lax.axis_index("s") * nc + lax.axis_index("c")
        span = pl.ds(wid * b_per_w, b_per_w)
        c1 = pltpu.async_copy(x1_hbm.at[span], idx1_v, sem1)
        c2 = pltpu.async_copy(x2_hbm.at[span], idx2_v, sem2)
        c1.wait()
        g1 = pltpu.async_copy(ip_hbm.at[idx1_v], p1_v, sem1)
        c2.wait()
        g2 = pltpu.async_copy(ip_hbm.at[idx2_v], p2_v, sem2)
        g1.wait()
        g2.wait()
        for i in range(b_per_w // lanes):
            sl = pl.ds(i * lanes, lanes)
            s = p1_v[sl] + p2_v[sl]
            p1_v[sl] = jnp.where(s >= TWO_PI, s - TWO_PI, s)
        pltpu.sync_copy(p1_v, out_hbm.at[span])

    return sc_phi


_NBUF = 4


def _tc_dense(phi, output_phases, block_rows):
    batch = phi.shape[0]
    k = output_phases.shape[0]
    nsteps = batch // block_rows
    assert nsteps % _NBUF == 0

    def body(phi_ref, op_ref, out_ref, bufs, sems):
        i = pl.program_id(0)
        n = pl.num_programs(0)
        slot = lax.rem(i, _NBUF)

        @pl.when(i >= _NBUF)
        def _():
            pltpu.make_async_copy(
                bufs.at[slot], out_ref.at[pl.ds(0, block_rows)], sems.at[slot]
            ).wait()

        d = jnp.abs(phi_ref[...] - op_ref[...])
        bufs[slot] = -jnp.minimum(d, TWO_PI - d)
        pltpu.make_async_copy(
            bufs.at[slot], out_ref.at[pl.ds(i * block_rows, block_rows)],
            sems.at[slot],
        ).start()

        @pl.when(i == n - 1)
        def _():
            for s in range(_NBUF):
                pltpu.make_async_copy(
                    bufs.at[s], out_ref.at[pl.ds(0, block_rows)], sems.at[s]
                ).wait()

    return pl.pallas_call(
        body,
        grid=(nsteps,),
        in_specs=[
            pl.BlockSpec((block_rows, 1), lambda i: (i, 0)),
            pl.BlockSpec((1, k), lambda i: (0, 0)),
        ],
        out_specs=pl.BlockSpec(memory_space=pl.ANY),
        out_shape=jax.ShapeDtypeStruct((batch, k), jnp.float32),
        scratch_shapes=[
            pltpu.VMEM((_NBUF, block_rows, k), jnp.float32),
            pltpu.SemaphoreType.DMA((_NBUF,)),
        ],
    )(phi.reshape(batch, 1), output_phases.reshape(1, k))


def kernel(x1, x2, input_phases, output_phases):
    x1 = x1.astype(jnp.int32)
    x2 = x2.astype(jnp.int32)
    input_phases = input_phases.astype(jnp.float32)
    output_phases = output_phases.astype(jnp.float32)
    batch = x1.shape[0]
    sc_phi = _make_sc_phi(batch)
    phi = sc_phi(x1, x2, input_phases)
    return _tc_dense(phi, output_phases, block_rows=2048)
